# Initial kernel scaffold; baseline (speedup 1.0000x reference)
import jax
import jax.numpy as jnp
from jax.experimental import pallas as pl


def kernel(coordinates, receivers, senders, mu, beta):
    raise NotImplementedError("write your pallas kernel here")


# speedup vs baseline: 3.7225x; 3.7225x over previous
import functools

import jax
import jax.numpy as jnp
from jax import lax
from jax.experimental import pallas as pl
from jax.experimental.pallas import tpu as pltpu
from jax.experimental.pallas import tpu_sc as plsc

N_NODES = 100000
N_EDGES = 1600000
N_BASIS = 32
CUTOFF = 5.0

_NC = 2
_NS = 16
_L = 16
_NW = _NC * _NS
_EW = N_EDGES // _NW
_SUP = 2000
_NSUP = _EW // _SUP
_GC = 80
_NG = _SUP // _GC
_NGRP = _SUP // _L


@functools.partial(
    pl.kernel,
    out_type=jax.ShapeDtypeStruct((N_EDGES,), jnp.float32),
    mesh=plsc.VectorSubcoreMesh(core_axis_name="c", subcore_axis_name="s"),
    scratch_types=[
        pltpu.VMEM((_SUP,), jnp.int32),
        pltpu.VMEM((_SUP,), jnp.int32),
        pltpu.VMEM((_SUP,), jnp.float32),
        pltpu.VMEM((_SUP,), jnp.float32),
        pltpu.VMEM((_SUP,), jnp.float32),
        pltpu.VMEM((_SUP,), jnp.float32),
        pltpu.VMEM((_SUP,), jnp.float32),
        pltpu.VMEM((_SUP,), jnp.float32),
        pltpu.VMEM((_SUP,), jnp.float32),
        pltpu.SemaphoreType.DMA,
    ],
)
def _sc_d2(cx_hbm, cy_hbm, cz_hbm, recv_hbm, send_hbm, d2_hbm,
           ridx_v, sidx_v, rx_v, ry_v, rz_v, sx_v, sy_v, sz_v, d2_v, sem):

    wid = lax.axis_index("s") * _NC + lax.axis_index("c")
    base = wid * _EW

    def super_body(s, carry):
        off = base + s * _SUP
        pltpu.sync_copy(recv_hbm.at[pl.ds(off, _SUP)], ridx_v)
        pltpu.sync_copy(send_hbm.at[pl.ds(off, _SUP)], sidx_v)

        def gather_body(g, c):
            gb = g * _GC
            ri = ridx_v.at[pl.ds(gb, _GC)]
            si = sidx_v.at[pl.ds(gb, _GC)]
            sl = pl.ds(gb, _GC)
            cps = [
                pltpu.async_copy(cx_hbm.at[ri], rx_v.at[sl], sem),
                pltpu.async_copy(cy_hbm.at[ri], ry_v.at[sl], sem),
                pltpu.async_copy(cz_hbm.at[ri], rz_v.at[sl], sem),
                pltpu.async_copy(cx_hbm.at[si], sx_v.at[sl], sem),
                pltpu.async_copy(cy_hbm.at[si], sy_v.at[sl], sem),
                pltpu.async_copy(cz_hbm.at[si], sz_v.at[sl], sem),
            ]
            for cp in cps:
                cp.wait()
            return c

        lax.fori_loop(0, _NG, gather_body, 0, unroll=False)

        def comp_body(i, c):
            sl = pl.ds(i * _L, _L)
            dx = rx_v[sl] - sx_v[sl]
            dy = ry_v[sl] - sy_v[sl]
            dz = rz_v[sl] - sz_v[sl]
            d2_v[sl] = dx * dx + dy * dy + dz * dz
            return c

        lax.fori_loop(0, _NGRP, comp_body, 0, unroll=False)
        pltpu.sync_copy(d2_v, d2_hbm.at[pl.ds(off, _SUP)])
        return carry

    lax.fori_loop(0, _NSUP, super_body, 0, unroll=False)


_ROWS = N_EDGES // 4
_BLK = 2000


def _tc_rbf(d2_ref, exp_ref, mu_ref, beta_ref, out_ref):
    r = jnp.sqrt(d2_ref[:])
    r128 = jnp.dot(r, exp_ref[:], preferred_element_type=jnp.float32,
                   precision=jax.lax.Precision.HIGHEST)
    u = r128 * (1.0 / CUTOFF)
    phi = 1.0 + u * u * u * (-10.0 + u * (15.0 - 6.0 * u))
    z = jnp.exp(-r128) - mu_ref[:]
    out_ref[:] = jnp.exp(-beta_ref[:] * z * z) * phi


def kernel(coordinates, receivers, senders, mu, beta):
    coords3 = coordinates.reshape(N_NODES, 3)
    cx = coords3[:, 0]
    cy = coords3[:, 1]
    cz = coords3[:, 2]
    d2 = _sc_d2(cx, cy, cz, receivers, senders)

    expand = jnp.kron(jnp.eye(4, dtype=jnp.float32),
                      jnp.ones((1, N_BASIS), jnp.float32))
    mu128 = jnp.tile(mu, (1, 4))
    beta128 = jnp.tile(beta, (1, 4))

    rbf = pl.pallas_call(
        _tc_rbf,
        grid=(_ROWS // _BLK,),
        in_specs=[
            pl.BlockSpec((_BLK, 4), lambda i: (i, 0)),
            pl.BlockSpec((4, 128), lambda i: (0, 0)),
            pl.BlockSpec((1, 128), lambda i: (0, 0)),
            pl.BlockSpec((1, 128), lambda i: (0, 0)),
        ],
        out_specs=pl.BlockSpec((_BLK, 128), lambda i: (i, 0)),
        out_shape=jax.ShapeDtypeStruct((_ROWS, 128), jnp.float32),
    )(d2.reshape(_ROWS, 4), expand, mu128, beta128)
    return rbf.reshape(N_EDGES, N_BASIS)


# speedup vs baseline: 5.1024x; 1.3707x over previous
import functools

import jax
import jax.numpy as jnp
from jax import lax
from jax.experimental import pallas as pl
from jax.experimental.pallas import tpu as pltpu
from jax.experimental.pallas import tpu_sc as plsc

N_NODES = 100000
N_EDGES = 1600000
N_BASIS = 32
CUTOFF = 5.0

_NC = 2
_NS = 16
_L = 16
_NW = _NC * _NS
_EW = N_EDGES // _NW
_SUP = 2000
_NSUP = _EW // _SUP
_GC = 80
_NG = _SUP // _GC
_NGRP = _SUP // _L


@functools.partial(
    pl.kernel,
    out_type=[jax.ShapeDtypeStruct((N_EDGES,), jnp.float32),
              jax.ShapeDtypeStruct((N_EDGES,), jnp.float32)],
    mesh=plsc.VectorSubcoreMesh(core_axis_name="c", subcore_axis_name="s"),
    scratch_types=[
        pltpu.VMEM((_SUP,), jnp.int32),
        pltpu.VMEM((_SUP,), jnp.int32),
        pltpu.VMEM((_SUP,), jnp.float32),
        pltpu.VMEM((_SUP,), jnp.float32),
        pltpu.VMEM((_SUP,), jnp.float32),
        pltpu.VMEM((_SUP,), jnp.float32),
        pltpu.VMEM((_SUP,), jnp.float32),
        pltpu.VMEM((_SUP,), jnp.float32),
        pltpu.VMEM((_SUP,), jnp.float32),
        pltpu.VMEM((_SUP,), jnp.float32),
        pltpu.SemaphoreType.DMA,
    ],
)
def _sc_edge(cx_hbm, cy_hbm, cz_hbm, recv_hbm, send_hbm, t_hbm, phi_hbm,
             ridx_v, sidx_v, rx_v, ry_v, rz_v, sx_v, sy_v, sz_v,
             t_v, phi_v, sem):

    wid = lax.axis_index("s") * _NC + lax.axis_index("c")
    base = wid * _EW

    def super_body(s, carry):
        off = base + s * _SUP
        pltpu.sync_copy(recv_hbm.at[pl.ds(off, _SUP)], ridx_v)
        pltpu.sync_copy(send_hbm.at[pl.ds(off, _SUP)], sidx_v)

        def gather_body(g, c):
            gb = g * _GC
            ri = ridx_v.at[pl.ds(gb, _GC)]
            si = sidx_v.at[pl.ds(gb, _GC)]
            sl = pl.ds(gb, _GC)
            cps = [
                pltpu.async_copy(cx_hbm.at[ri], rx_v.at[sl], sem),
                pltpu.async_copy(cy_hbm.at[ri], ry_v.at[sl], sem),
                pltpu.async_copy(cz_hbm.at[ri], rz_v.at[sl], sem),
                pltpu.async_copy(cx_hbm.at[si], sx_v.at[sl], sem),
                pltpu.async_copy(cy_hbm.at[si], sy_v.at[sl], sem),
                pltpu.async_copy(cz_hbm.at[si], sz_v.at[sl], sem),
            ]
            for cp in cps:
                cp.wait()
            return c

        lax.fori_loop(0, _NG, gather_body, 0, unroll=False)

        def comp_body(i, c):
            sl = pl.ds(i * _L, _L)
            dx = rx_v[sl] - sx_v[sl]
            dy = ry_v[sl] - sy_v[sl]
            dz = rz_v[sl] - sz_v[sl]
            d2 = dx * dx + dy * dy + dz * dz
            ybits = jnp.int32(0x5F3759DF) - lax.shift_right_logical(
                lax.bitcast_convert_type(d2, jnp.int32), 1)
            y = lax.bitcast_convert_type(ybits, jnp.float32)
            y = y * (1.5 - 0.5 * ((d2 * y) * y))
            y = y * (1.5 - 0.5 * ((d2 * y) * y))
            y = y * (1.5 - 0.5 * ((d2 * y) * y))
            r = d2 * y
            t_v[sl] = jnp.exp(-r)
            u = r * (1.0 / CUTOFF)
            u2 = u * u
            phi_v[sl] = 1.0 + u2 * u * (-10.0 + 15.0 * u - 6.0 * u2)
            return c

        lax.fori_loop(0, _NGRP, comp_body, 0, unroll=False)
        pltpu.sync_copy(t_v, t_hbm.at[pl.ds(off, _SUP)])
        pltpu.sync_copy(phi_v, phi_hbm.at[pl.ds(off, _SUP)])
        return carry

    lax.fori_loop(0, _NSUP, super_body, 0, unroll=False)


_NBLK = 125
_BR = 100
_BE = _BR * 128


def _tc_rbf(t_ref, phi_ref, mu_ref, beta_ref, out_ref):
    t3 = lax.broadcast_in_dim(t_ref[0], (_BR, 128, N_BASIS), (0, 1))
    p3 = lax.broadcast_in_dim(phi_ref[0], (_BR, 128, N_BASIS), (0, 1))
    z = t3 - mu_ref[:]
    out_ref[:] = jnp.exp(-beta_ref[:] * z * z) * p3


def kernel(coordinates, receivers, senders, mu, beta):
    coords3 = coordinates.reshape(N_NODES, 3)
    cx = coords3[:, 0]
    cy = coords3[:, 1]
    cz = coords3[:, 2]
    t, phi = _sc_edge(cx, cy, cz, receivers, senders)

    rbf = pl.pallas_call(
        _tc_rbf,
        grid=(_NBLK,),
        in_specs=[
            pl.BlockSpec((1, _BR, 128), lambda i: (i, 0, 0)),
            pl.BlockSpec((1, _BR, 128), lambda i: (i, 0, 0)),
            pl.BlockSpec((1, 1, N_BASIS), lambda i: (0, 0, 0)),
            pl.BlockSpec((1, 1, N_BASIS), lambda i: (0, 0, 0)),
        ],
        out_specs=pl.BlockSpec((_BR, 128, N_BASIS), lambda i: (i, 0, 0)),
        out_shape=jax.ShapeDtypeStruct((_NBLK * _BR, 128, N_BASIS),
                                       jnp.float32),
    )(t.reshape(_NBLK, _BR, 128), phi.reshape(_NBLK, _BR, 128),
      mu.reshape(1, 1, N_BASIS), beta.reshape(1, 1, N_BASIS))
    return rbf.reshape(N_EDGES, N_BASIS)


# speedup vs baseline: 6.8574x; 1.3440x over previous
import functools

import jax
import jax.numpy as jnp
from jax import lax
from jax.experimental import pallas as pl
from jax.experimental.pallas import tpu as pltpu
from jax.experimental.pallas import tpu_sc as plsc

N_NODES = 100000
N_EDGES = 1600000
N_BASIS = 32
CUTOFF = 5.0

_NC = 2
_NS = 16
_L = 16
_NW = _NC * _NS
_EW = N_EDGES // _NW
_SUP = 2000
_NSUP = _EW // _SUP
_GC = 80
_NG = _SUP // _GC
_NGRP = _SUP // _L


@functools.partial(
    pl.kernel,
    out_type=[jax.ShapeDtypeStruct((N_EDGES,), jnp.float32),
              jax.ShapeDtypeStruct((N_EDGES,), jnp.float32)],
    mesh=plsc.VectorSubcoreMesh(core_axis_name="c", subcore_axis_name="s"),
    scratch_types=[
        pltpu.VMEM((_SUP,), jnp.int32),
        pltpu.VMEM((_SUP,), jnp.int32),
        pltpu.VMEM((_SUP,), jnp.float32),
        pltpu.VMEM((_SUP,), jnp.float32),
        pltpu.VMEM((_SUP,), jnp.float32),
        pltpu.VMEM((_SUP,), jnp.float32),
        pltpu.VMEM((_SUP,), jnp.float32),
        pltpu.VMEM((_SUP,), jnp.float32),
        pltpu.VMEM((_SUP,), jnp.float32),
        pltpu.VMEM((_SUP,), jnp.float32),
        pltpu.SemaphoreType.DMA,
    ],
)
def _sc_edge(cx_hbm, cy_hbm, cz_hbm, recv_hbm, send_hbm, t_hbm, phi_hbm,
             ridx_v, sidx_v, rx_v, ry_v, rz_v, sx_v, sy_v, sz_v,
             t_v, phi_v, sem):

    wid = lax.axis_index("s") * _NC + lax.axis_index("c")
    base = wid * _EW

    def super_body(s, carry):
        off = base + s * _SUP
        pltpu.sync_copy(recv_hbm.at[pl.ds(off, _SUP)], ridx_v)
        pltpu.sync_copy(send_hbm.at[pl.ds(off, _SUP)], sidx_v)

        def fire_body(g, c):
            gb = g * _GC
            sl = pl.ds(gb, _GC)
            ri = ridx_v.at[sl]
            si = sidx_v.at[sl]
            pltpu.async_copy(cx_hbm.at[ri], rx_v.at[sl], sem)
            pltpu.async_copy(cy_hbm.at[ri], ry_v.at[sl], sem)
            pltpu.async_copy(cz_hbm.at[ri], rz_v.at[sl], sem)
            pltpu.async_copy(cx_hbm.at[si], sx_v.at[sl], sem)
            pltpu.async_copy(cy_hbm.at[si], sy_v.at[sl], sem)
            pltpu.async_copy(cz_hbm.at[si], sz_v.at[sl], sem)
            return c

        lax.fori_loop(0, _NG, fire_body, 0, unroll=False)

        def drain_body(g, c):
            gb = g * _GC
            sl = pl.ds(gb, _GC)
            ri = ridx_v.at[sl]
            si = sidx_v.at[sl]
            pltpu.make_async_copy(cx_hbm.at[ri], rx_v.at[sl], sem).wait()
            pltpu.make_async_copy(cy_hbm.at[ri], ry_v.at[sl], sem).wait()
            pltpu.make_async_copy(cz_hbm.at[ri], rz_v.at[sl], sem).wait()
            pltpu.make_async_copy(cx_hbm.at[si], sx_v.at[sl], sem).wait()
            pltpu.make_async_copy(cy_hbm.at[si], sy_v.at[sl], sem).wait()
            pltpu.make_async_copy(cz_hbm.at[si], sz_v.at[sl], sem).wait()
            return c

        lax.fori_loop(0, _NG, drain_body, 0, unroll=False)

        def comp_body(i, c):
            sl = pl.ds(i * _L, _L)
            dx = rx_v[sl] - sx_v[sl]
            dy = ry_v[sl] - sy_v[sl]
            dz = rz_v[sl] - sz_v[sl]
            d2 = dx * dx + dy * dy + dz * dz
            ybits = jnp.int32(0x5F3759DF) - lax.shift_right_logical(
                lax.bitcast_convert_type(d2, jnp.int32), 1)
            y = lax.bitcast_convert_type(ybits, jnp.float32)
            y = y * (1.5 - 0.5 * ((d2 * y) * y))
            y = y * (1.5 - 0.5 * ((d2 * y) * y))
            y = y * (1.5 - 0.5 * ((d2 * y) * y))
            r = d2 * y
            t_v[sl] = jnp.exp(-r)
            u = r * (1.0 / CUTOFF)
            u2 = u * u
            phi_v[sl] = 1.0 + u2 * u * (-10.0 + 15.0 * u - 6.0 * u2)
            return c

        lax.fori_loop(0, _NGRP, comp_body, 0, unroll=False)
        pltpu.sync_copy(t_v, t_hbm.at[pl.ds(off, _SUP)])
        pltpu.sync_copy(phi_v, phi_hbm.at[pl.ds(off, _SUP)])
        return carry

    lax.fori_loop(0, _NSUP, super_body, 0, unroll=False)


_NBLK = 125
_BR = 100
_BE = _BR * 128


def _tc_rbf(t_ref, phi_ref, mu_ref, beta_ref, out_ref):
    t3 = lax.broadcast_in_dim(t_ref[0], (_BR, 128, N_BASIS), (0, 1))
    p3 = lax.broadcast_in_dim(phi_ref[0], (_BR, 128, N_BASIS), (0, 1))
    z = t3 - mu_ref[:]
    out_ref[:] = jnp.exp(-beta_ref[:] * z * z) * p3


def kernel(coordinates, receivers, senders, mu, beta):
    coords3 = coordinates.reshape(N_NODES, 3)
    t, phi = _sc_edge(coords3[:, 0], coords3[:, 1], coords3[:, 2],
                      receivers, senders)

    rbf = pl.pallas_call(
        _tc_rbf,
        grid=(_NBLK,),
        in_specs=[
            pl.BlockSpec((1, _BR, 128), lambda i: (i, 0, 0)),
            pl.BlockSpec((1, _BR, 128), lambda i: (i, 0, 0)),
            pl.BlockSpec((1, 1, N_BASIS), lambda i: (0, 0, 0)),
            pl.BlockSpec((1, 1, N_BASIS), lambda i: (0, 0, 0)),
        ],
        out_specs=pl.BlockSpec((_BR, 128, N_BASIS), lambda i: (i, 0, 0)),
        out_shape=jax.ShapeDtypeStruct((_NBLK * _BR, 128, N_BASIS),
                                       jnp.float32),
    )(t.reshape(_NBLK, _BR, 128), phi.reshape(_NBLK, _BR, 128),
      mu.reshape(1, 1, N_BASIS), beta.reshape(1, 1, N_BASIS))
    return rbf.reshape(N_EDGES, N_BASIS)
